# baseline (speedup 1.0000x reference)
import functools

import jax
import jax.numpy as jnp
from jax import lax
from jax.experimental import pallas as pl
from jax.experimental.pallas import tpu as pltpu
from jax.experimental.pallas import tpu_sc as plsc

def _lane_perm(v, idx):
    dnums = lax.GatherDimensionNumbers(
        offset_dims=(), collapsed_slice_dims=(0,), start_index_map=(0,))
    return lax.gather(v, idx[:, None], dnums, slice_sizes=(1,),
                      mode=lax.GatherScatterMode.PROMISE_IN_BOUNDS)


E = 8
W = 8192
D = 1024
NC = 2
NS = 16
NW = NC * NS
WIN_PER = W // NW
CW = 16
ROWS = 2 * CW
NCHUNK = WIN_PER // CW
ZROWS = CW
NBUF = 3


def _sc_body(x_hbm, eidx_hbm, out_hbm, eidx_v, srcl, dstl, zbuf,
             buf0, buf1, buf2, zsem0, zsem1, gsem0, gsem1, gsem2, dsem0,
             dsem1, dsem2):
    bufs = (buf0, buf1, buf2)
    gsems = (gsem0, gsem1, gsem2)
    dsems = (dsem0, dsem1, dsem2)
    zsems = (zsem0, zsem1)
    wid = lax.axis_index("s") * NC + lax.axis_index("c")
    base = wid * WIN_PER

    zero16 = jnp.zeros((16,), jnp.float32)
    for r in range(ZROWS):
        for c in range(D // 16):
            zbuf[r, pl.ds(c * 16, 16)] = zero16

    def fire_zero(k):
        return [pltpu.async_copy(
                    zbuf,
                    out_hbm.at[pl.ds(e * W + base + k * CW, CW)],
                    zsems[k % 2])
                for e in range(E)]

    zcp = [None] * NCHUNK
    zcp[0] = fire_zero(0)
    if NCHUNK > 1:
        zcp[1] = fire_zero(1)

    pltpu.sync_copy(eidx_hbm.at[pl.ds(2 * base, 2 * WIN_PER)], eidx_v)

    lane = lax.iota(jnp.int32, 16)
    partner_perm = lane ^ 1
    even = (lane & 1) == 0
    for k in range(NCHUNK):
        for c in range(2):
            pos = 32 * k + 16 * c + lane
            ev = eidx_v[pl.ds(32 * k + 16 * c, 16)]
            partner = _lane_perm(ev, partner_perm)
            dup = (ev == partner) & even
            wg = base + (pos >> 1)
            srcl[k, pl.ds(16 * c, 16)] = (2 * base + pos
                                          + jnp.where(dup, 1, 0))
            dstl[k, pl.ds(16 * c, 16)] = ev * W + wg

    gcp = [None] * NCHUNK
    dcp = [None] * NCHUNK
    for k in range(NBUF - 1):
        gcp[k] = pltpu.async_copy(x_hbm.at[srcl.at[k]], bufs[k % NBUF],
                                  gsems[k % NBUF])

    for k in range(NCHUNK):
        s = k % NBUF
        gcp[k].wait()
        for c in zcp[k]:
            c.wait()
        dcp[k] = pltpu.async_copy(bufs[s], out_hbm.at[dstl.at[k]], dsems[s])
        if k + 2 < NCHUNK:
            zcp[k + 2] = fire_zero(k + 2)
        nk = k + NBUF - 1
        if nk < NCHUNK:
            ns = nk % NBUF
            if nk >= NBUF:
                dcp[nk - NBUF].wait()
            gcp[nk] = pltpu.async_copy(x_hbm.at[srcl.at[nk]], bufs[ns],
                                       gsems[ns])
    for k in range(max(0, NCHUNK - NBUF), NCHUNK):
        dcp[k].wait()


@jax.jit
def _dispatch(x_flat, eidx_flat):
    mesh = plsc.VectorSubcoreMesh(core_axis_name="c", subcore_axis_name="s")
    run = pl.kernel(
        _sc_body,
        mesh=mesh,
        out_type=jax.ShapeDtypeStruct((E * W, D), jnp.float32),
        scratch_types=[
            pltpu.VMEM((2 * WIN_PER,), jnp.int32),
            pltpu.VMEM((NCHUNK, ROWS), jnp.int32),
            pltpu.VMEM((NCHUNK, ROWS), jnp.int32),
            pltpu.VMEM((ZROWS, D), jnp.float32),
            pltpu.VMEM((ROWS, D), jnp.float32),
            pltpu.VMEM((ROWS, D), jnp.float32),
            pltpu.VMEM((ROWS, D), jnp.float32),
        ] + [pltpu.SemaphoreType.DMA] * 8,
    )
    return run(x_flat, eidx_flat)


def kernel(isp_per_win, expert_indices, num_experts):
    b, w, k, d = isp_per_win.shape
    x_flat = isp_per_win.reshape(b * w * k, d)
    eidx_flat = expert_indices.reshape(-1)
    out = _dispatch(x_flat, eidx_flat)
    return out.reshape(E, b * w, d)


# speedup vs baseline: 1.0581x; 1.0581x over previous
import functools

import jax
import jax.numpy as jnp
from jax import lax
from jax.experimental import pallas as pl
from jax.experimental.pallas import tpu as pltpu
from jax.experimental.pallas import tpu_sc as plsc

def _lane_perm(v, idx):
    dnums = lax.GatherDimensionNumbers(
        offset_dims=(), collapsed_slice_dims=(0,), start_index_map=(0,))
    return lax.gather(v, idx[:, None], dnums, slice_sizes=(1,),
                      mode=lax.GatherScatterMode.PROMISE_IN_BOUNDS)


E = 8
W = 8192
D = 1024
NC = 2
NS = 16
NW = NC * NS
WIN_PER = W // NW
CW = 16
ROWS = 2 * CW
NCHUNK = WIN_PER // CW
ZROWS = CW
NBUF = 3


def _sc_body(x_hbm, eidx_hbm, out_hbm, eidx_v, srcl, dstl, zbuf,
             buf0, buf1, buf2, zsem0, zsem1, gsem0, gsem1, gsem2, dsem0,
             dsem1, dsem2):
    bufs = (buf0, buf1, buf2)
    gsems = (gsem0, gsem1, gsem2)
    dsems = (dsem0, dsem1, dsem2)
    zsems = (zsem0, zsem1)
    wid = lax.axis_index("s") * NC + lax.axis_index("c")
    base = wid * WIN_PER

    zero16 = jnp.zeros((16,), jnp.float32)
    for r in range(ZROWS):
        for c in range(D // 16):
            zbuf[r, pl.ds(c * 16, 16)] = zero16

    zcopies = []
    for e in range(E):
        for s in range(WIN_PER // ZROWS):
            zcopies.append(
                pltpu.async_copy(
                    zbuf,
                    out_hbm.at[pl.ds(e * W + base + s * ZROWS, ZROWS)],
                    zsems[s % 2]))

    pltpu.sync_copy(eidx_hbm.at[pl.ds(2 * base, 2 * WIN_PER)], eidx_v)

    lane = lax.iota(jnp.int32, 16)
    partner_perm = lane ^ 1
    even = (lane & 1) == 0
    for k in range(NCHUNK):
        for c in range(2):
            pos = 32 * k + 16 * c + lane
            ev = eidx_v[pl.ds(32 * k + 16 * c, 16)]
            partner = _lane_perm(ev, partner_perm)
            dup = (ev == partner) & even
            wg = base + (pos >> 1)
            srcl[k, pl.ds(16 * c, 16)] = (2 * base + pos
                                          + jnp.where(dup, 1, 0))
            dstl[k, pl.ds(16 * c, 16)] = ev * W + wg

    gcp = [None] * NCHUNK
    dcp = [None] * NCHUNK
    for k in range(NBUF - 1):
        gcp[k] = pltpu.async_copy(x_hbm.at[srcl.at[k]], bufs[k % NBUF],
                                  gsems[k % NBUF])

    for c in zcopies:
        c.wait()

    for k in range(NCHUNK):
        s = k % NBUF
        gcp[k].wait()
        dcp[k] = pltpu.async_copy(bufs[s], out_hbm.at[dstl.at[k]], dsems[s])
        nk = k + NBUF - 1
        if nk < NCHUNK:
            ns = nk % NBUF
            if nk >= NBUF:
                dcp[nk - NBUF].wait()
            gcp[nk] = pltpu.async_copy(x_hbm.at[srcl.at[nk]], bufs[ns],
                                       gsems[ns])
    for k in range(max(0, NCHUNK - NBUF), NCHUNK):
        dcp[k].wait()


@jax.jit
def _dispatch(x_flat, eidx_flat):
    mesh = plsc.VectorSubcoreMesh(core_axis_name="c", subcore_axis_name="s")
    run = pl.kernel(
        _sc_body,
        mesh=mesh,
        out_type=jax.ShapeDtypeStruct((E * W, D), jnp.float32),
        scratch_types=[
            pltpu.VMEM((2 * WIN_PER,), jnp.int32),
            pltpu.VMEM((NCHUNK, ROWS), jnp.int32),
            pltpu.VMEM((NCHUNK, ROWS), jnp.int32),
            pltpu.VMEM((ZROWS, D), jnp.float32),
            pltpu.VMEM((ROWS, D), jnp.float32),
            pltpu.VMEM((ROWS, D), jnp.float32),
            pltpu.VMEM((ROWS, D), jnp.float32),
        ] + [pltpu.SemaphoreType.DMA] * 8,
    )
    return run(x_flat, eidx_flat)


def kernel(isp_per_win, expert_indices, num_experts):
    b, w, k, d = isp_per_win.shape
    x_flat = isp_per_win.reshape(b * w * k, d)
    eidx_flat = expert_indices.reshape(-1)
    out = _dispatch(x_flat, eidx_flat)
    return out.reshape(E, b * w, d)
